# baseline (speedup 1.0000x reference)
import functools

import jax
import jax.numpy as jnp
from jax import lax
from jax.experimental import pallas as pl
from jax.experimental.pallas import tpu as pltpu
from jax.experimental.pallas import tpu_sc as plsc

D = 64
K = 10
NC = 2
NS = 16
NW = NC * NS
CHUNK_OUT = 8
CHUNK_IDX = CHUNK_OUT * K
LANES = 16
DV = D // LANES
NBUF = 4

NT = 1000000
DB = 128
NBLK = 999936 // DB

_mesh = plsc.VectorSubcoreMesh(core_axis_name="c", subcore_axis_name="s")


@functools.cache
def _make_repack():
    nblk_w = -(-(NBLK + 1) // NW)
    nblk_pad = -(-nblk_w // 4) * 4

    NR = 4

    @functools.partial(
        pl.kernel,
        mesh=_mesh,
        compiler_params=pltpu.CompilerParams(needs_layout_passes=False),
        out_type=jax.ShapeDtypeStruct((NT // 2, 2 * D), jnp.float32),
        scratch_types=[
            [pltpu.VMEM((D, DB), jnp.float32) for _ in range(NR)],
            [pltpu.VMEM((DB // 2, 2 * D), jnp.float32) for _ in range(2)],
            [pltpu.SemaphoreType.DMA for _ in range(NR)],
            [pltpu.SemaphoreType.DMA for _ in range(2)],
        ],
    )
    def k(tt_hbm, t64p_hbm, x_hbm, bufa, bufb, rsems, wsems):
        cid = lax.axis_index("c")
        sid = lax.axis_index("s")
        wid = sid * NC + cid

        def blk_of(j):
            return jnp.minimum(wid + NW * j, NBLK - 1)

        def start_read(j, a):
            off = pl.multiple_of(blk_of(j) * DB, 128)
            pltpu.async_copy(
                tt_hbm.at[pl.ds(0, D), pl.ds(off, DB)], bufa[a], rsems[a]
            )

        def wait_read(a):
            pltpu.make_async_copy(
                tt_hbm.at[pl.ds(0, D), pl.ds(0, DB)], bufa[a], rsems[a]
            ).wait()

        def start_write(j, w):
            off = pl.multiple_of(blk_of(j) * (DB // 2), 8)
            pltpu.async_copy(
                bufb[w], x_hbm.at[pl.ds(off, DB // 2)], wsems[w]
            )

        def wait_write(w):
            pltpu.make_async_copy(
                bufb[w], x_hbm.at[pl.ds(0, DB // 2)], wsems[w]
            ).wait()

        dio = jax.lax.broadcasted_iota(jnp.int32, (LANES,), 0)

        def transpose_cols(a, w, ncols):
            def prow(r2, carry):
                for h in range(2):
                    cs = jnp.full((LANES,), 2 * r2 + h, dtype=jnp.int32)
                    for d in range(DV):
                        v = plsc.load_gather(bufa[a], [dio + d * LANES, cs])
                        bufb[w][r2, pl.ds(h * D + d * LANES, LANES)] = v
                return carry

            lax.fori_loop(0, ncols // 2, prow, 0)

        def step(j, bb, wait_w):
            a = bb % NR
            w = bb % 2
            wait_read(a)
            start_read(j + NR - 1, (bb + NR - 1) % NR)
            if wait_w:
                wait_write(w)
            transpose_cols(a, w, DB)
            start_write(j, w)

        for a in range(NR - 1):
            start_read(a, a)
        for j in range(NR):
            step(j, j, wait_w=j >= 2)

        def body(i, carry):
            for bb in range(NR):
                step(NR + NR * i + bb, bb, wait_w=True)
            return carry

        lax.fori_loop(0, (nblk_pad - NR) // NR, body, 0)
        for w in range(2):
            wait_write(w)
        for a in range(NR - 1):
            wait_read((nblk_pad + a) % NR)

        @pl.when(wid == 0)
        def _():
            pltpu.sync_copy(
                t64p_hbm, x_hbm.at[pl.ds(NBLK * DB // 2, D // 2)]
            )

    return k


@functools.cache
def _make_gather(n_out: int):
    per_w = n_out // NW
    nchunk = per_w // CHUNK_OUT
    assert per_w * NW == n_out and nchunk * CHUNK_OUT == per_w
    assert nchunk % NBUF == 0 and nchunk >= 3 * NBUF

    @functools.partial(
        pl.kernel,
        mesh=_mesh,
        compiler_params=pltpu.CompilerParams(use_tc_tiling_on_sc=False),
        out_type=jax.ShapeDtypeStruct((n_out, D), jnp.float32),
        scratch_types=[
            pltpu.VMEM((nchunk, CHUNK_IDX), jnp.int32),
            [pltpu.VMEM((CHUNK_IDX, D), jnp.float32) for _ in range(NBUF)],
            [pltpu.VMEM((CHUNK_OUT, D), jnp.float32) for _ in range(NBUF)],
            [pltpu.SemaphoreType.DMA for _ in range(NBUF)],
            [pltpu.SemaphoreType.DMA for _ in range(NBUF)],
        ],
    )
    def k(idx_hbm, table_hbm, out_hbm, idx_v, rows, outs, gsems, osems):
        wid = lax.axis_index("s") * NC + lax.axis_index("c")
        out_base = wid * per_w

        def start_gather(c, b):
            pltpu.async_copy(table_hbm.at[idx_v.at[c]], rows[b], gsems[b])

        def wait_gather(b):
            pltpu.make_async_copy(table_hbm.at[idx_v.at[0]], rows[b], gsems[b]).wait()

        def compute(c, b):
            r = rows[b]
            o_v = outs[b]
            for o in range(CHUNK_OUT):
                base = o * K
                for d in range(DV):
                    sl = pl.ds(d * LANES, LANES)
                    acc = r[base, sl]
                    for kk in range(1, K):
                        acc = acc + r[base + kk, sl]
                    o_v[o, sl] = acc * jnp.float32(1.0 / K)
            pltpu.async_copy(
                o_v, out_hbm.at[pl.ds(out_base + c * CHUNK_OUT, CHUNK_OUT)],
                osems[b],
            )

        def wait_outstore(b):
            pltpu.make_async_copy(
                outs[b], out_hbm.at[pl.ds(out_base, CHUNK_OUT)], osems[b]
            ).wait()

        pltpu.sync_copy(idx_hbm.at[wid], idx_v)

        for b in range(NBUF):
            start_gather(b, b)
        for b in range(NBUF):
            wait_gather(b)
            compute(b, b)
            start_gather(b + NBUF, b)

        def outer(i, carry):
            for b in range(NBUF):
                c = NBUF + i * NBUF + b
                wait_gather(b)
                wait_outstore(b)
                compute(c, b)
                start_gather(jnp.minimum(c + NBUF, nchunk - 1), b)
            return carry

        lax.fori_loop(0, nchunk // NBUF - 1, outer, 0)

        for b in range(NBUF):
            wait_gather(b)
            wait_outstore(b)

    return k


def kernel(channel_items, table):
    B, C, Kk = channel_items.shape
    n_out = B * C
    idx = channel_items.astype(jnp.int32).reshape(
        NW, n_out * Kk // (NW * CHUNK_IDX), CHUNK_IDX
    )
    t64p = table[NBLK * DB : NT].reshape(D // 2, 2 * D)
    x = _make_repack()(table.T, t64p)
    out = _make_gather(n_out)(idx, x.reshape(NT, D))
    return out.reshape(B, C, D)


# speedup vs baseline: 1.9192x; 1.9192x over previous
import functools

import jax
import jax.numpy as jnp
from jax import lax
from jax.experimental import pallas as pl
from jax.experimental.pallas import tpu as pltpu
from jax.experimental.pallas import tpu_sc as plsc

D = 64
K = 10
NC = 2
NS = 16
NW = NC * NS
CHUNK_OUT = 8
CHUNK_IDX = CHUNK_OUT * K
LANES = 16
DV = D // LANES
NBUF = 4


@functools.cache
def _make_kernel(n_out: int):
    per_w = n_out // NW
    nchunk = per_w // CHUNK_OUT
    assert per_w * NW == n_out and nchunk * CHUNK_OUT == per_w
    assert nchunk % NBUF == 0 and nchunk >= 3 * NBUF
    mesh = plsc.VectorSubcoreMesh(core_axis_name="c", subcore_axis_name="s")

    @functools.partial(
        pl.kernel,
        mesh=mesh,
        compiler_params=pltpu.CompilerParams(use_tc_tiling_on_sc=False),
        out_type=jax.ShapeDtypeStruct((n_out, D), jnp.float32),
        scratch_types=[
            pltpu.VMEM((nchunk, CHUNK_IDX), jnp.int32),
            [pltpu.VMEM((CHUNK_IDX, D), jnp.float32) for _ in range(NBUF)],
            [pltpu.VMEM((CHUNK_OUT, D), jnp.float32) for _ in range(NBUF)],
            [pltpu.SemaphoreType.DMA for _ in range(NBUF)],
            [pltpu.SemaphoreType.DMA for _ in range(NBUF)],
        ],
    )
    def k(idx_hbm, table_hbm, out_hbm, idx_v, rows, outs, gsems, osems):
        wid = lax.axis_index("s") * NC + lax.axis_index("c")
        out_base = wid * per_w

        def start_gather(c, b):
            pltpu.async_copy(table_hbm.at[idx_v.at[c]], rows[b], gsems[b])

        def wait_gather(b):
            pltpu.make_async_copy(table_hbm.at[idx_v.at[0]], rows[b], gsems[b]).wait()

        def compute(c, b):
            r = rows[b]
            o_v = outs[b]
            for o in range(CHUNK_OUT):
                base = o * K
                for d in range(DV):
                    sl = pl.ds(d * LANES, LANES)
                    acc = r[base, sl]
                    for kk in range(1, K):
                        acc = acc + r[base + kk, sl]
                    o_v[o, sl] = acc * jnp.float32(1.0 / K)
            pltpu.async_copy(
                o_v, out_hbm.at[pl.ds(out_base + c * CHUNK_OUT, CHUNK_OUT)], osems[b]
            )

        def wait_outstore(b):
            pltpu.make_async_copy(
                outs[b], out_hbm.at[pl.ds(out_base, CHUNK_OUT)], osems[b]
            ).wait()

        pltpu.sync_copy(idx_hbm.at[wid], idx_v)

        for b in range(NBUF):
            start_gather(b, b)
        for b in range(NBUF):
            wait_gather(b)
            compute(b, b)
            start_gather(b + NBUF, b)

        def outer(i, carry):
            for b in range(NBUF):
                c = NBUF + i * NBUF + b
                wait_gather(b)
                wait_outstore(b)
                compute(c, b)
                start_gather(jnp.minimum(c + NBUF, nchunk - 1), b)
            return carry

        lax.fori_loop(0, nchunk // NBUF - 1, outer, 0)

        for b in range(NBUF):
            wait_gather(b)
            wait_outstore(b)

    return k


def kernel(channel_items, table):
    B, C, Kk = channel_items.shape
    n_out = B * C
    idx = channel_items.astype(jnp.int32).reshape(
        NW, n_out * Kk // (NW * CHUNK_IDX), CHUNK_IDX
    )
    out = _make_kernel(n_out)(idx, table)
    return out.reshape(B, C, D)
